# Initial kernel scaffold; baseline (speedup 1.0000x reference)
import functools

import jax
import jax.numpy as jnp
from jax import lax
from jax.experimental import pallas as pl
from jax.experimental.pallas import tpu as pltpu
from jax.experimental.pallas import tpu_sc as plsc

N_X = 100000
D_X = 64
N_E = 3200000
D_E = 16

CX = 400
CE = 1000


def _build():
    info = plsc.get_sparse_core_info()
    nc, ns = info.num_cores, info.num_subcores
    nw = nc * ns

    n_chunks_x = N_X // CX
    n_chunks_e = N_E // CE
    x_iters = -(-n_chunks_x // nw)
    e_iters = n_chunks_e // nw

    mesh = plsc.VectorSubcoreMesh(core_axis_name="c", subcore_axis_name="s")

    @functools.partial(
        pl.kernel,
        mesh=mesh,
        out_type=[
            jax.ShapeDtypeStruct((N_X, D_X), jnp.float32),
            jax.ShapeDtypeStruct((N_E, D_E), jnp.float32),
        ],
        scratch_types=[
            pltpu.VMEM((CX,), jnp.int32),
            pltpu.VMEM((CX, D_X), jnp.float32),
            pltpu.VMEM((CE,), jnp.int32),
            pltpu.VMEM((CE, D_E), jnp.float32),
            pltpu.SemaphoreType.DMA,
        ],
    )
    def emb(x_hbm, e_hbm, xtab_hbm, etab_hbm, outx_hbm, oute_hbm,
            idx_x, rows_x, idx_e, rows_e, sem):
        wid = lax.axis_index("s") * nc + lax.axis_index("c")

        def x_body(i, carry):
            cid = wid + i * nw

            @pl.when(cid < n_chunks_x)
            def _():
                base = cid * CX
                pltpu.sync_copy(x_hbm.at[pl.ds(base, CX)], idx_x)
                pltpu.async_copy(xtab_hbm.at[idx_x], rows_x, sem).wait()
                pltpu.sync_copy(rows_x, outx_hbm.at[pl.ds(base, CX)])

            return carry

        lax.fori_loop(0, x_iters, x_body, 0)

        def e_body(i, carry):
            cid = wid + i * nw
            base = cid * CE
            pltpu.sync_copy(e_hbm.at[pl.ds(base, CE)], idx_e)
            pltpu.async_copy(etab_hbm.at[idx_e], rows_e, sem).wait()
            pltpu.sync_copy(rows_e, oute_hbm.at[pl.ds(base, CE)])
            return carry

        lax.fori_loop(0, e_iters, e_body, 0)

    return emb


_EMB = _build()


def kernel(x, edge_attr, embed_x_table, embed_edge_table):
    x = x.astype(jnp.int32)
    edge_attr = edge_attr.astype(jnp.int32)
    out_x, out_edge = _EMB(x, edge_attr, embed_x_table, embed_edge_table)
    return (out_x, out_edge)


# speedup vs baseline: 6.5703x; 6.5703x over previous
import functools

import jax
import jax.numpy as jnp
from jax import lax
from jax.experimental import pallas as pl
from jax.experimental.pallas import tpu as pltpu
from jax.experimental.pallas import tpu_sc as plsc

N_X = 100000
D_X = 64
N_E = 3200000
D_E = 16

CX = 400
CE = 1000


def _build():
    info = plsc.get_sparse_core_info()
    nc, ns = info.num_cores, info.num_subcores
    nw = nc * ns

    n_chunks_x = N_X // CX
    n_chunks_e = N_E // CE
    x_iters = -(-n_chunks_x // nw)
    e_iters = n_chunks_e // nw

    mesh = plsc.VectorSubcoreMesh(core_axis_name="c", subcore_axis_name="s")

    @functools.partial(
        pl.kernel,
        mesh=mesh,
        out_type=[
            jax.ShapeDtypeStruct((N_X, D_X), jnp.float32),
            jax.ShapeDtypeStruct((N_E, D_E), jnp.float32),
        ],
        scratch_types=[
            pltpu.VMEM((CX,), jnp.int32),
            pltpu.VMEM((CX, D_X), jnp.float32),
            pltpu.VMEM((CE,), jnp.int32),
            pltpu.VMEM((CE, D_E), jnp.float32),
            pltpu.SemaphoreType.DMA,
        ],
        compiler_params=pltpu.CompilerParams(use_tc_tiling_on_sc=False),
    )
    def emb(x_hbm, e_hbm, xtab_hbm, etab_hbm, outx_hbm, oute_hbm,
            idx_x, rows_x, idx_e, rows_e, sem):
        wid = lax.axis_index("s") * nc + lax.axis_index("c")

        def x_body(i, carry):
            cid = wid + i * nw

            @pl.when(cid < n_chunks_x)
            def _():
                base = cid * CX
                pltpu.sync_copy(x_hbm.at[pl.ds(base, CX)], idx_x)
                pltpu.async_copy(xtab_hbm.at[idx_x], rows_x, sem).wait()
                pltpu.sync_copy(rows_x, outx_hbm.at[pl.ds(base, CX)])

            return carry

        lax.fori_loop(0, x_iters, x_body, 0)

        def e_body(i, carry):
            cid = wid + i * nw
            base = cid * CE
            pltpu.sync_copy(e_hbm.at[pl.ds(base, CE)], idx_e)
            pltpu.async_copy(etab_hbm.at[idx_e], rows_e, sem).wait()
            pltpu.sync_copy(rows_e, oute_hbm.at[pl.ds(base, CE)])
            return carry

        lax.fori_loop(0, e_iters, e_body, 0)

    return emb


_EMB = _build()


def kernel(x, edge_attr, embed_x_table, embed_edge_table):
    x = x.astype(jnp.int32)
    edge_attr = edge_attr.astype(jnp.int32)
    out_x, out_edge = _EMB(x, edge_attr, embed_x_table, embed_edge_table)
    return (out_x, out_edge)


# speedup vs baseline: 7.8322x; 1.1921x over previous
import functools

import jax
import jax.numpy as jnp
from jax import lax
from jax.experimental import pallas as pl
from jax.experimental.pallas import tpu as pltpu
from jax.experimental.pallas import tpu_sc as plsc

N_X = 100000
D_X = 64
N_E = 3200000
D_E = 16

CX = 400
CE = 1000


def _build():
    info = plsc.get_sparse_core_info()
    nc, ns = info.num_cores, info.num_subcores
    nw = nc * ns

    n_chunks_x = N_X // CX
    n_chunks_e = N_E // CE
    x_iters = -(-n_chunks_x // nw)
    e_iters = n_chunks_e // nw

    mesh = plsc.VectorSubcoreMesh(core_axis_name="c", subcore_axis_name="s")

    @functools.partial(
        pl.kernel,
        mesh=mesh,
        out_type=[
            jax.ShapeDtypeStruct((N_X, D_X), jnp.float32),
            jax.ShapeDtypeStruct((N_E, D_E), jnp.float32),
        ],
        scratch_types=[
            pltpu.VMEM((CX,), jnp.int32),
            pltpu.VMEM((CX, D_X), jnp.float32),
            pltpu.VMEM((CE,), jnp.int32),
            pltpu.VMEM((CE, D_E), jnp.float32),
            pltpu.VMEM_SHARED((512, D_E), jnp.float32),
            pltpu.SemaphoreType.DMA,
        ],
        compiler_params=pltpu.CompilerParams(use_tc_tiling_on_sc=False),
    )
    def emb(x_hbm, e_hbm, xtab_hbm, etab_hbm, outx_hbm, oute_hbm,
            idx_x, rows_x, idx_e, rows_e, etab_v, sem):
        wid = lax.axis_index("s") * nc + lax.axis_index("c")
        @pl.when(lax.axis_index("s") == 0)
        def _():
            pltpu.sync_copy(etab_hbm, etab_v)

        plsc.subcore_barrier()

        def x_body(i, carry):
            cid = wid + i * nw

            @pl.when(cid < n_chunks_x)
            def _():
                base = cid * CX
                pltpu.sync_copy(x_hbm.at[pl.ds(base, CX)], idx_x)
                pltpu.async_copy(xtab_hbm.at[idx_x], rows_x, sem).wait()
                pltpu.sync_copy(rows_x, outx_hbm.at[pl.ds(base, CX)])

            return carry

        lax.fori_loop(0, x_iters, x_body, 0)

        def e_body(i, carry):
            cid = wid + i * nw
            base = cid * CE
            pltpu.sync_copy(e_hbm.at[pl.ds(base, CE)], idx_e)
            pltpu.async_copy(etab_v.at[idx_e], rows_e, sem).wait()
            pltpu.sync_copy(rows_e, oute_hbm.at[pl.ds(base, CE)])
            return carry

        lax.fori_loop(0, e_iters, e_body, 0)

    return emb


_EMB = _build()


def kernel(x, edge_attr, embed_x_table, embed_edge_table):
    x = x.astype(jnp.int32)
    edge_attr = edge_attr.astype(jnp.int32)
    out_x, out_edge = _EMB(x, edge_attr, embed_x_table, embed_edge_table)
    return (out_x, out_edge)


# speedup vs baseline: 8.2593x; 1.0545x over previous
import functools

import jax
import jax.numpy as jnp
from jax import lax
from jax.experimental import pallas as pl
from jax.experimental.pallas import tpu as pltpu
from jax.experimental.pallas import tpu_sc as plsc

N_X = 100000
D_X = 64
N_E = 3200000
D_E = 16
V_E = 512

CX = 400
CE = 1000


def _build():
    info = plsc.get_sparse_core_info()
    nc, ns = info.num_cores, info.num_subcores
    nw = nc * ns

    ncx = N_X // CX
    nce = N_E // CE
    tx = -(-ncx // nw)
    te = nce // nw
    tx_pairs = tx // 2
    te_pairs = te // 2

    mesh = plsc.VectorSubcoreMesh(core_axis_name="c", subcore_axis_name="s")

    @functools.partial(
        pl.kernel,
        mesh=mesh,
        out_type=[
            jax.ShapeDtypeStruct((N_X, D_X), jnp.float32),
            jax.ShapeDtypeStruct((N_E, D_E), jnp.float32),
        ],
        scratch_types=[
            pltpu.VMEM((CX,), jnp.int32),
            pltpu.VMEM((CX,), jnp.int32),
            pltpu.VMEM((CX, D_X), jnp.float32),
            pltpu.VMEM((CX, D_X), jnp.float32),
            pltpu.VMEM((CE,), jnp.int32),
            pltpu.VMEM((CE,), jnp.int32),
            pltpu.VMEM((CE, D_E), jnp.float32),
            pltpu.VMEM((CE, D_E), jnp.float32),
            pltpu.VMEM_SHARED((V_E, D_E), jnp.float32),
            pltpu.SemaphoreType.DMA,
            pltpu.SemaphoreType.DMA,
            pltpu.SemaphoreType.DMA,
            pltpu.SemaphoreType.DMA,
            pltpu.SemaphoreType.DMA,
            pltpu.SemaphoreType.DMA,
        ],
        compiler_params=pltpu.CompilerParams(use_tc_tiling_on_sc=False),
    )
    def emb(x_hbm, e_hbm, xtab_hbm, etab_hbm, outx_hbm, oute_hbm,
            idx_x0, idx_x1, rows_x0, rows_x1,
            idx_e0, idx_e1, rows_e0, rows_e1, etab_s,
            s_i0, s_i1, s_g0, s_g1, s_o0, s_o1):
        wid = lax.axis_index("s") * nc + lax.axis_index("c")

        @pl.when(lax.axis_index("s") == 0)
        def _():
            pltpu.sync_copy(etab_hbm, etab_s)

        plsc.subcore_barrier()

        def xb(t):
            return (wid + t * nw) * CX

        def xvalid(t):
            return (wid + t * nw) < ncx

        pltpu.async_copy(x_hbm.at[pl.ds(xb(0), CX)], idx_x0, s_i0)
        pltpu.async_copy(x_hbm.at[pl.ds(xb(1), CX)], idx_x1, s_i1)

        def x_body(j, carry):
            t0 = 2 * j
            t1 = 2 * j + 1

            @pl.when(j > 0)
            def _():
                pltpu.make_async_copy(
                    rows_x0, outx_hbm.at[pl.ds(xb(t0 - 2), CX)], s_o0).wait()

            pltpu.make_async_copy(
                x_hbm.at[pl.ds(xb(t0), CX)], idx_x0, s_i0).wait()
            pltpu.async_copy(xtab_hbm.at[idx_x0], rows_x0, s_g0)

            @pl.when(j > 0)
            def _():
                pltpu.make_async_copy(
                    rows_x1, outx_hbm.at[pl.ds(xb(t1 - 2), CX)], s_o1).wait()

            @pl.when(xvalid(t1))
            def _():
                pltpu.make_async_copy(
                    x_hbm.at[pl.ds(xb(t1), CX)], idx_x1, s_i1).wait()
                pltpu.async_copy(xtab_hbm.at[idx_x1], rows_x1, s_g1)

            pltpu.make_async_copy(xtab_hbm.at[idx_x0], rows_x0, s_g0).wait()
            pltpu.async_copy(rows_x0, outx_hbm.at[pl.ds(xb(t0), CX)], s_o0)

            @pl.when(j < tx_pairs - 1)
            def _():
                pltpu.async_copy(x_hbm.at[pl.ds(xb(t0 + 2), CX)], idx_x0, s_i0)

            @pl.when(xvalid(t1))
            def _():
                pltpu.make_async_copy(
                    xtab_hbm.at[idx_x1], rows_x1, s_g1).wait()
                pltpu.async_copy(rows_x1, outx_hbm.at[pl.ds(xb(t1), CX)], s_o1)

            @pl.when(xvalid(t1 + 2))
            def _():
                pltpu.async_copy(x_hbm.at[pl.ds(xb(t1 + 2), CX)], idx_x1, s_i1)

            return carry

        lax.fori_loop(0, tx_pairs, x_body, 0)

        pltpu.make_async_copy(
            rows_x0, outx_hbm.at[pl.ds(xb(tx - 2), CX)], s_o0).wait()

        @pl.when(xvalid(tx - 1))
        def _():
            pltpu.make_async_copy(
                rows_x1, outx_hbm.at[pl.ds(xb(tx - 1), CX)], s_o1).wait()

        def ebase(t):
            return (wid + t * nw) * CE

        pltpu.async_copy(e_hbm.at[pl.ds(ebase(0), CE)], idx_e0, s_i0)
        pltpu.async_copy(e_hbm.at[pl.ds(ebase(1), CE)], idx_e1, s_i1)

        def e_body(j, carry):
            t0 = 2 * j
            t1 = 2 * j + 1

            @pl.when(j > 0)
            def _():
                pltpu.make_async_copy(
                    rows_e0, oute_hbm.at[pl.ds(ebase(t0 - 2), CE)], s_o0).wait()

            pltpu.make_async_copy(
                e_hbm.at[pl.ds(ebase(t0), CE)], idx_e0, s_i0).wait()
            pltpu.async_copy(etab_s.at[idx_e0], rows_e0, s_g0)

            @pl.when(j > 0)
            def _():
                pltpu.make_async_copy(
                    rows_e1, oute_hbm.at[pl.ds(ebase(t1 - 2), CE)], s_o1).wait()

            pltpu.make_async_copy(
                e_hbm.at[pl.ds(ebase(t1), CE)], idx_e1, s_i1).wait()
            pltpu.async_copy(etab_s.at[idx_e1], rows_e1, s_g1)

            pltpu.make_async_copy(etab_s.at[idx_e0], rows_e0, s_g0).wait()
            pltpu.async_copy(rows_e0, oute_hbm.at[pl.ds(ebase(t0), CE)], s_o0)

            @pl.when(j < te_pairs - 1)
            def _():
                pltpu.async_copy(e_hbm.at[pl.ds(ebase(t0 + 2), CE)], idx_e0, s_i0)

            pltpu.make_async_copy(etab_s.at[idx_e1], rows_e1, s_g1).wait()
            pltpu.async_copy(rows_e1, oute_hbm.at[pl.ds(ebase(t1), CE)], s_o1)

            @pl.when(j < te_pairs - 1)
            def _():
                pltpu.async_copy(e_hbm.at[pl.ds(ebase(t1 + 2), CE)], idx_e1, s_i1)

            return carry

        lax.fori_loop(0, te_pairs, e_body, 0)

        pltpu.make_async_copy(
            rows_e0, oute_hbm.at[pl.ds(ebase(te - 2), CE)], s_o0).wait()
        pltpu.make_async_copy(
            rows_e1, oute_hbm.at[pl.ds(ebase(te - 1), CE)], s_o1).wait()

    return emb


_EMB = _build()


def kernel(x, edge_attr, embed_x_table, embed_edge_table):
    x = x.astype(jnp.int32)
    edge_attr = edge_attr.astype(jnp.int32)
    out_x, out_edge = _EMB(x, edge_attr, embed_x_table, embed_edge_table)
    return (out_x, out_edge)
